# baseline (speedup 1.0000x reference)
import functools

import jax
import jax.numpy as jnp
from jax import lax
from jax.experimental import pallas as pl
from jax.experimental.pallas import tpu as pltpu
from jax.experimental.pallas import tpu_sc as plsc

N = 10000
E = 320000
D = 128
NP = 10240

NC = 2
NS = 16
L = 16
KB = 128
CH = 40
SLOW_CORE = 0

EP = 327680
NBLK_ES = EP // (NC * NS) // KB
NBLK_FS = EP // NS // KB
RA = NP // NS

RB = 512
G = NP // RB

_mesh = plsc.VectorSubcoreMesh(core_axis_name="c", subcore_axis_name="s",
                               num_cores=NC, num_subcores=NS)

_f32 = jnp.float32
_i32 = jnp.int32

_sc_params = pltpu.CompilerParams(needs_layout_passes=False,
                                  use_tc_tiling_on_sc=False)


def _splat16(v):
    return jnp.full((L,), v, dtype=_i32)


def _sc_deg(dst_es, ew_es):
    @functools.partial(
        pl.kernel,
        out_type=jax.ShapeDtypeStruct((NC * NS * NP,), _f32),
        mesh=_mesh,
        scratch_types=[
            pltpu.VMEM((NBLK_ES, KB), _i32),
            pltpu.VMEM((NBLK_ES, KB), _f32),
            pltpu.VMEM((NP,), _f32),
        ],
        compiler_params=_sc_params,
    )
    def k(dst_hbm, ew_hbm, out_hbm, dsts, ews, deg):
        c = lax.axis_index("c")
        s = lax.axis_index("s")
        wid = c * NS + s
        pltpu.sync_copy(dst_hbm.at[wid], dsts)
        pltpu.sync_copy(ew_hbm.at[wid], ews)

        @pl.loop(0, NP // L)
        def _(i):
            deg[pl.ds(i * L, L)] = jnp.zeros((L,), _f32)

        @pl.loop(0, NBLK_ES)
        def _(j):
            for k16 in range(KB // L):
                sl = pl.ds(k16 * L, L)
                plsc.addupdate_scatter(deg, [dsts[j, sl]], ews[j, sl])

        pltpu.sync_copy(deg, out_hbm.at[pl.ds(wid * NP, NP)])

    return k(dst_es, ew_es)


def _sc_agg(g, src_slab, dst_slab, ew_slab, w, feat_split, kb):
    A = EP // kb // (NC * NS)

    NB = 4
    NG = CH // NB

    @functools.partial(
        pl.kernel,
        out_type=jax.ShapeDtypeStruct((NC * NP, w), _f32),
        mesh=_mesh,
        scratch_types=[
            pltpu.VMEM((CH, kb), _i32),
            pltpu.VMEM((CH, kb), _i32),
            pltpu.VMEM((CH, kb), _f32),
        ] + [pltpu.VMEM((kb, w), _f32)] * NB
          + [pltpu.SemaphoreType.DMA] * (2 * NB)
          + [pltpu.VMEM_SHARED((NP, w), _f32)],
        compiler_params=_sc_params,
    )
    def k(g_hbm, src_hbm, dst_hbm, ew_hbm, out_hbm, srcs, dsts, ews, *rest):
        rows = rest[:NB]
        gs = rest[NB:2 * NB]
        ss = rest[2 * NB:3 * NB]
        acc = rest[3 * NB]
        c = lax.axis_index("c")
        s = lax.axis_index("s")
        if feat_split:
            base = s * (2 * A)
            nph = (2 * A) // CH
        else:
            a_lo, a_hi = A // 2, (3 * A) // 2
            slow = c == SLOW_CORE
            base = jnp.where(slow, s * a_lo, NS * a_lo + s * a_hi)
            nph = jnp.where(slow, a_lo // CH, a_hi // CH)

        @pl.loop(0, kb)
        def _(e):
            for f in range(w // L):
                rows[0][e, pl.ds(f * L, L)] = jnp.zeros((L,), _f32)

        for i in range(RA // kb):
            pltpu.sync_copy(rows[0], acc.at[pl.ds(s * RA + i * kb, kb)])
        plsc.subcore_barrier()

        def gather(j, b):
            pltpu.make_async_copy(g_hbm.at[srcs.at[j]], rows[b], gs[b]).start()

        def gather_wait(j, b):
            pltpu.make_async_copy(g_hbm.at[srcs.at[j]], rows[b], gs[b]).wait()

        def scatter(j, b):
            pltpu.make_async_copy(rows[b], acc.at[dsts.at[j]],
                                  ss[b]).start(add=True)

        def scatter_wait(j, b):
            pltpu.make_async_copy(rows[b], acc.at[dsts.at[j]], ss[b]).wait()

        def scale(j, b):
            @plsc.parallel_loop(0, kb, unroll=4)
            def _(e):
                sp = plsc.load_gather(ews, [_splat16(j), _splat16(e)])
                for f in range(w // L):
                    sl = pl.ds(f * L, L)
                    rows[b][e, sl] = rows[b][e, sl] * sp

        @pl.loop(0, nph)
        def _(ph):
            pltpu.sync_copy(src_hbm.at[pl.ds(base + ph * CH, CH)], srcs)
            pltpu.sync_copy(dst_hbm.at[pl.ds(base + ph * CH, CH)], dsts)
            pltpu.sync_copy(ew_hbm.at[pl.ds(base + ph * CH, CH)], ews)

            if feat_split:
                off = jnp.full((L,), c * NP, dtype=_i32)

                @pl.loop(0, CH)
                def _(j):
                    for k16 in range(kb // L):
                        sl = pl.ds(k16 * L, L)
                        srcs[j, sl] = srcs[j, sl] + off

            for b in range(NB):
                gather(b, b)

            @pl.loop(0, NG)
            def _(g):
                for b in range(NB):
                    j = g * NB + b
                    gather_wait(j, b)
                    scale(j, b)
                    scatter(j, b)

                @pl.when(g < NG - 1)
                def _():
                    for b in range(NB):
                        j = g * NB + b
                        jn = j + NB
                        scatter_wait(j, b)
                        gather(jn, b)

            for b in range(NB):
                scatter_wait((NG - 1) * NB + b, b)

        plsc.subcore_barrier()
        pltpu.sync_copy(acc.at[pl.ds(s * RA, RA)],
                        out_hbm.at[pl.ds(c * NP + s * RA, RA)])

    return k(g, src_slab, dst_slab, ew_slab)


def _rowspec(w):
    return pl.BlockSpec((RB, w), lambda i: (i, 0))


def _fullspec(shape):
    nd = len(shape)
    return pl.BlockSpec(shape, lambda i, _nd=nd: (0,) * _nd)


def _dot_t(a, w):
    return lax.dot_general(a, w, (((1,), (1,)), ((), ())),
                           preferred_element_type=_f32)


def _tc_l1(x, W1, degp):
    def body(x_ref, w_ref, dp_ref, dis_ref, g1_ref):
        m = _dot_t(x_ref[...], w_ref[...])
        ones = jnp.ones((NC * NS, 1), _f32)
        d = 1.0 + lax.dot_general(dp_ref[...], ones, (((0,), (0,)), ((), ())),
                                  preferred_element_type=_f32)
        dis = lax.rsqrt(d)
        dis_ref[...] = dis
        g1_ref[...] = m * dis

    return pl.pallas_call(
        body,
        grid=(G,),
        in_specs=[_rowspec(D), _fullspec(W1.shape),
                  pl.BlockSpec((NC * NS, RB), lambda i: (0, i))],
        out_specs=[_rowspec(1), _rowspec(64)],
        out_shape=[jax.ShapeDtypeStruct((NP, 1), _f32),
                   jax.ShapeDtypeStruct((NP, 64), _f32)],
    )(x, W1, degp)


def _p_specs(w):
    return [pl.BlockSpec((RB, w), lambda i: (i, 0)),
            pl.BlockSpec((RB, w), lambda i: (G + i, 0))]


def _tc_post1(P, g1, dis, b1):
    def body(p0_ref, p1_ref, g1_ref, dis_ref, b_ref, g2_ref):
        dis = dis_ref[...]
        a = dis * (p0_ref[...] + p1_ref[...] + g1_ref[...])
        g2_ref[...] = dis * jnp.maximum(a + b_ref[...][None, :], 0.0)

    return pl.pallas_call(
        body,
        grid=(G,),
        in_specs=_p_specs(64) + [_rowspec(64), _rowspec(1),
                                 _fullspec(b1.shape)],
        out_specs=_rowspec(64),
        out_shape=jax.ShapeDtypeStruct((NP, 64), _f32),
    )(P, P, g1, dis, b1)


def _tc_mid(P, g, dis, W, b, win, wout):
    def body(p0_ref, p1_ref, g_ref, dis_ref, w_ref, b_ref, o_ref):
        dis = dis_ref[...]
        a = dis * (p0_ref[...] + p1_ref[...] + g_ref[...])
        h = jnp.maximum(_dot_t(a, w_ref[...]) + b_ref[...][None, :], 0.0)
        o_ref[...] = dis * h

    return pl.pallas_call(
        body,
        grid=(G,),
        in_specs=_p_specs(win) + [_rowspec(win), _rowspec(1),
                                  _fullspec(W.shape), _fullspec(b.shape)],
        out_specs=_rowspec(wout),
        out_shape=jax.ShapeDtypeStruct((NP, wout), _f32),
    )(P, P, g, dis, W, b)


def _tc_l3(P, g3, dis, W3, b3):
    def body(p0_ref, p1_ref, g_ref, dis_ref, w_ref, b_ref, o_ref):
        dis = dis_ref[...]
        a = dis * (p0_ref[...] + p1_ref[...] + g_ref[...])
        h = jnp.maximum(_dot_t(a, w_ref[...]) + b_ref[...][None, :], 0.0)
        g4 = dis * h
        o_ref[0] = g4[:, :128]
        o_ref[1] = g4[:, 128:]

    return pl.pallas_call(
        body,
        grid=(G,),
        in_specs=_p_specs(128) + [_rowspec(128), _rowspec(1),
                                  _fullspec(W3.shape), _fullspec(b3.shape)],
        out_specs=pl.BlockSpec((2, RB, 128), lambda i: (0, i, 0)),
        out_shape=jax.ShapeDtypeStruct((2, NP, 128), _f32),
    )(P, P, g3, dis, W3, b3)


def _tc_head(Q, g4s, dis, W4, b4, Wf1, bf1, Wf2, bf2):
    def body(q0_ref, q1_ref, g4_ref, dis_ref, w4_ref, b4_ref,
             wf1_ref, bf1_ref, wf2_ref, bf2_ref, o_ref):
        dis = dis_ref[...]
        a_lo = dis * (q0_ref[...] + g4_ref[0])
        a_hi = dis * (q1_ref[...] + g4_ref[1])
        a4 = jnp.concatenate([a_lo, a_hi], axis=1)
        h4 = jnp.maximum(_dot_t(a4, w4_ref[...]) + b4_ref[...][None, :], 0.0)
        z = jnp.maximum(_dot_t(h4, wf1_ref[...]) + bf1_ref[...][None, :], 0.0)
        o_ref[...] = (jnp.sum(z * wf2_ref[...], axis=1, keepdims=True)
                      + bf2_ref[0])

    return pl.pallas_call(
        body,
        grid=(G,),
        in_specs=_p_specs(128) +
        [pl.BlockSpec((2, RB, 128), lambda i: (0, i, 0)),
         _rowspec(1),
         _fullspec(W4.shape), _fullspec(b4.shape),
         _fullspec(Wf1.shape), _fullspec(bf1.shape),
         _fullspec(Wf2.shape), _fullspec(bf2.shape)],
        out_specs=_rowspec(1),
        out_shape=jax.ShapeDtypeStruct((NP, 1), _f32),
    )(Q, Q, g4s, dis, W4, b4, Wf1, bf1, Wf2, bf2)


def kernel(x, edge_index, edge_attr, W1, b1, W2, b2, W3, b3, W4, b4,
           Wf1, bf1, Wf2, bf2):
    pad = EP - E
    src = jnp.concatenate([edge_index[0], jnp.zeros((pad,), _i32)])
    dst = jnp.concatenate([edge_index[1], jnp.zeros((pad,), _i32)])
    ew = jnp.concatenate([edge_attr, jnp.zeros((pad,), _f32)])

    src_es = src.reshape(NC * NS, NBLK_ES, KB)
    dst_es = dst.reshape(NC * NS, NBLK_ES, KB)
    ew_es = ew.reshape(NC * NS, NBLK_ES, KB)
    src_f128 = src.reshape(EP // KB, KB)
    dst_f128 = dst.reshape(EP // KB, KB)
    ew_f128 = ew.reshape(EP // KB, KB)
    src_f64 = src.reshape(2 * EP // KB, KB // 2)
    dst_f64 = dst.reshape(2 * EP // KB, KB // 2)
    ew_f64 = ew.reshape(2 * EP // KB, KB // 2)

    xp = jnp.pad(x, ((0, NP - N), (0, 0)))

    degp = _sc_deg(dst_es, ew_es).reshape(NC * NS, NP)
    dis, g1 = _tc_l1(xp, W1, degp)

    P = _sc_agg(g1, src_f128, dst_f128, ew_f128, 64, False, KB)
    g2 = _tc_post1(P, g1, dis, b1)

    P = _sc_agg(g2, src_f128, dst_f128, ew_f128, 64, False, KB)
    g3 = _tc_mid(P, g2, dis, W2, b2, 64, 128)

    P = _sc_agg(g3, src_f64, dst_f64, ew_f64, 128, False, KB // 2)
    g4s = _tc_l3(P, g3, dis, W3, b3)

    Q = _sc_agg(g4s.reshape(2 * NP, 128), src_f64, dst_f64, ew_f64, 128,
                True, KB // 2)
    out = _tc_head(Q, g4s, dis, W4, b4, Wf1, bf1, Wf2, bf2)
    return out[:N]


# speedup vs baseline: 1.0059x; 1.0059x over previous
import functools

import jax
import jax.numpy as jnp
from jax import lax
from jax.experimental import pallas as pl
from jax.experimental.pallas import tpu as pltpu
from jax.experimental.pallas import tpu_sc as plsc

N = 10000
E = 320000
D = 128
NP = 10240

NC = 2
NS = 16
L = 16
KB = 128
CH = 40
SLOW_CORE = 1

EP = 327680
NBLK_ES = EP // (NC * NS) // KB
NBLK_FS = EP // NS // KB
RA = NP // NS

RB = 512
G = NP // RB

_mesh = plsc.VectorSubcoreMesh(core_axis_name="c", subcore_axis_name="s",
                               num_cores=NC, num_subcores=NS)

_f32 = jnp.float32
_i32 = jnp.int32

_sc_params = pltpu.CompilerParams(needs_layout_passes=False,
                                  use_tc_tiling_on_sc=False)


def _splat16(v):
    return jnp.full((L,), v, dtype=_i32)


def _sc_deg(dst_es, ew_es):
    @functools.partial(
        pl.kernel,
        out_type=jax.ShapeDtypeStruct((NC * NS * NP,), _f32),
        mesh=_mesh,
        scratch_types=[
            pltpu.VMEM((NBLK_ES, KB), _i32),
            pltpu.VMEM((NBLK_ES, KB), _f32),
            pltpu.VMEM((NP,), _f32),
        ],
        compiler_params=_sc_params,
    )
    def k(dst_hbm, ew_hbm, out_hbm, dsts, ews, deg):
        c = lax.axis_index("c")
        s = lax.axis_index("s")
        wid = c * NS + s
        pltpu.sync_copy(dst_hbm.at[wid], dsts)
        pltpu.sync_copy(ew_hbm.at[wid], ews)

        @pl.loop(0, NP // L)
        def _(i):
            deg[pl.ds(i * L, L)] = jnp.zeros((L,), _f32)

        @pl.loop(0, NBLK_ES)
        def _(j):
            for k16 in range(KB // L):
                sl = pl.ds(k16 * L, L)
                plsc.addupdate_scatter(deg, [dsts[j, sl]], ews[j, sl])

        pltpu.sync_copy(deg, out_hbm.at[pl.ds(wid * NP, NP)])

    return k(dst_es, ew_es)


def _sc_agg(g, src_slab, dst_slab, ew_slab, w, feat_split, kb):
    A = EP // kb // (NC * NS)

    NB = 4
    NG = CH // NB

    @functools.partial(
        pl.kernel,
        out_type=jax.ShapeDtypeStruct((NC * NP, w), _f32),
        mesh=_mesh,
        scratch_types=[
            pltpu.VMEM((CH, kb), _i32),
            pltpu.VMEM((CH, kb), _i32),
            pltpu.VMEM((CH, kb), _f32),
        ] + [pltpu.VMEM((kb, w), _f32)] * NB
          + [pltpu.SemaphoreType.DMA] * (2 * NB)
          + [pltpu.VMEM_SHARED((NP, w), _f32)],
        compiler_params=_sc_params,
    )
    def k(g_hbm, src_hbm, dst_hbm, ew_hbm, out_hbm, srcs, dsts, ews, *rest):
        rows = rest[:NB]
        gs = rest[NB:2 * NB]
        ss = rest[2 * NB:3 * NB]
        acc = rest[3 * NB]
        c = lax.axis_index("c")
        s = lax.axis_index("s")
        if feat_split:
            base = s * (2 * A)
            nph = (2 * A) // CH
        else:
            a_lo, a_hi = A // 2, (3 * A) // 2
            slow = c == SLOW_CORE
            base = jnp.where(slow, s * a_lo, NS * a_lo + s * a_hi)
            nph = jnp.where(slow, a_lo // CH, a_hi // CH)

        @pl.loop(0, kb)
        def _(e):
            for f in range(w // L):
                rows[0][e, pl.ds(f * L, L)] = jnp.zeros((L,), _f32)

        for i in range(RA // kb):
            pltpu.sync_copy(rows[0], acc.at[pl.ds(s * RA + i * kb, kb)])
        plsc.subcore_barrier()

        def gather(j, b):
            pltpu.make_async_copy(g_hbm.at[srcs.at[j]], rows[b], gs[b]).start()

        def gather_wait(j, b):
            pltpu.make_async_copy(g_hbm.at[srcs.at[j]], rows[b], gs[b]).wait()

        def scatter(j, b):
            pltpu.make_async_copy(rows[b], acc.at[dsts.at[j]],
                                  ss[b]).start(add=True)

        def scatter_wait(j, b):
            pltpu.make_async_copy(rows[b], acc.at[dsts.at[j]], ss[b]).wait()

        def scale(j, b):
            @plsc.parallel_loop(0, kb, unroll=4)
            def _(e):
                sp = plsc.load_gather(ews, [_splat16(j), _splat16(e)])
                for f in range(w // L):
                    sl = pl.ds(f * L, L)
                    rows[b][e, sl] = rows[b][e, sl] * sp

        @pl.loop(0, nph)
        def _(ph):
            pltpu.sync_copy(src_hbm.at[pl.ds(base + ph * CH, CH)], srcs)
            pltpu.sync_copy(dst_hbm.at[pl.ds(base + ph * CH, CH)], dsts)
            pltpu.sync_copy(ew_hbm.at[pl.ds(base + ph * CH, CH)], ews)

            if feat_split:
                off = jnp.full((L,), c * NP, dtype=_i32)

                @pl.loop(0, CH)
                def _(j):
                    for k16 in range(kb // L):
                        sl = pl.ds(k16 * L, L)
                        srcs[j, sl] = srcs[j, sl] + off

            for b in range(NB):
                gather(b, b)

            @pl.loop(0, NG)
            def _(g):
                for b in range(NB):
                    j = g * NB + b
                    gather_wait(j, b)
                    scale(j, b)
                    scatter(j, b)

                @pl.when(g < NG - 1)
                def _():
                    for b in range(NB):
                        j = g * NB + b
                        jn = j + NB
                        scatter_wait(j, b)
                        gather(jn, b)

            for b in range(NB):
                scatter_wait((NG - 1) * NB + b, b)

        plsc.subcore_barrier()
        pltpu.sync_copy(acc.at[pl.ds(s * RA, RA)],
                        out_hbm.at[pl.ds(c * NP + s * RA, RA)])

    return k(g, src_slab, dst_slab, ew_slab)


def _rowspec(w):
    return pl.BlockSpec((RB, w), lambda i: (i, 0))


def _fullspec(shape):
    nd = len(shape)
    return pl.BlockSpec(shape, lambda i, _nd=nd: (0,) * _nd)


def _dot_t(a, w):
    return lax.dot_general(a, w, (((1,), (1,)), ((), ())),
                           preferred_element_type=_f32)


def _tc_l1(x, W1, degp):
    def body(x_ref, w_ref, dp_ref, dis_ref, g1_ref):
        m = _dot_t(x_ref[...], w_ref[...])
        ones = jnp.ones((NC * NS, 1), _f32)
        d = 1.0 + lax.dot_general(dp_ref[...], ones, (((0,), (0,)), ((), ())),
                                  preferred_element_type=_f32)
        dis = lax.rsqrt(d)
        dis_ref[...] = dis
        g1_ref[...] = m * dis

    return pl.pallas_call(
        body,
        grid=(G,),
        in_specs=[_rowspec(D), _fullspec(W1.shape),
                  pl.BlockSpec((NC * NS, RB), lambda i: (0, i))],
        out_specs=[_rowspec(1), _rowspec(64)],
        out_shape=[jax.ShapeDtypeStruct((NP, 1), _f32),
                   jax.ShapeDtypeStruct((NP, 64), _f32)],
    )(x, W1, degp)


def _p_specs(w):
    return [pl.BlockSpec((RB, w), lambda i: (i, 0)),
            pl.BlockSpec((RB, w), lambda i: (G + i, 0))]


def _tc_post1(P, g1, dis, b1):
    def body(p0_ref, p1_ref, g1_ref, dis_ref, b_ref, g2_ref):
        dis = dis_ref[...]
        a = dis * (p0_ref[...] + p1_ref[...] + g1_ref[...])
        g2_ref[...] = dis * jnp.maximum(a + b_ref[...][None, :], 0.0)

    return pl.pallas_call(
        body,
        grid=(G,),
        in_specs=_p_specs(64) + [_rowspec(64), _rowspec(1),
                                 _fullspec(b1.shape)],
        out_specs=_rowspec(64),
        out_shape=jax.ShapeDtypeStruct((NP, 64), _f32),
    )(P, P, g1, dis, b1)


def _tc_mid(P, g, dis, W, b, win, wout):
    def body(p0_ref, p1_ref, g_ref, dis_ref, w_ref, b_ref, o_ref):
        dis = dis_ref[...]
        a = dis * (p0_ref[...] + p1_ref[...] + g_ref[...])
        h = jnp.maximum(_dot_t(a, w_ref[...]) + b_ref[...][None, :], 0.0)
        o_ref[...] = dis * h

    return pl.pallas_call(
        body,
        grid=(G,),
        in_specs=_p_specs(win) + [_rowspec(win), _rowspec(1),
                                  _fullspec(W.shape), _fullspec(b.shape)],
        out_specs=_rowspec(wout),
        out_shape=jax.ShapeDtypeStruct((NP, wout), _f32),
    )(P, P, g, dis, W, b)


def _tc_l3(P, g3, dis, W3, b3):
    def body(p0_ref, p1_ref, g_ref, dis_ref, w_ref, b_ref, o_ref):
        dis = dis_ref[...]
        a = dis * (p0_ref[...] + p1_ref[...] + g_ref[...])
        h = jnp.maximum(_dot_t(a, w_ref[...]) + b_ref[...][None, :], 0.0)
        g4 = dis * h
        o_ref[0] = g4[:, :128]
        o_ref[1] = g4[:, 128:]

    return pl.pallas_call(
        body,
        grid=(G,),
        in_specs=_p_specs(128) + [_rowspec(128), _rowspec(1),
                                  _fullspec(W3.shape), _fullspec(b3.shape)],
        out_specs=pl.BlockSpec((2, RB, 128), lambda i: (0, i, 0)),
        out_shape=jax.ShapeDtypeStruct((2, NP, 128), _f32),
    )(P, P, g3, dis, W3, b3)


def _tc_head(Q, g4s, dis, W4, b4, Wf1, bf1, Wf2, bf2):
    def body(q0_ref, q1_ref, g4_ref, dis_ref, w4_ref, b4_ref,
             wf1_ref, bf1_ref, wf2_ref, bf2_ref, o_ref):
        dis = dis_ref[...]
        a_lo = dis * (q0_ref[...] + g4_ref[0])
        a_hi = dis * (q1_ref[...] + g4_ref[1])
        a4 = jnp.concatenate([a_lo, a_hi], axis=1)
        h4 = jnp.maximum(_dot_t(a4, w4_ref[...]) + b4_ref[...][None, :], 0.0)
        z = jnp.maximum(_dot_t(h4, wf1_ref[...]) + bf1_ref[...][None, :], 0.0)
        o_ref[...] = (jnp.sum(z * wf2_ref[...], axis=1, keepdims=True)
                      + bf2_ref[0])

    return pl.pallas_call(
        body,
        grid=(G,),
        in_specs=_p_specs(128) +
        [pl.BlockSpec((2, RB, 128), lambda i: (0, i, 0)),
         _rowspec(1),
         _fullspec(W4.shape), _fullspec(b4.shape),
         _fullspec(Wf1.shape), _fullspec(bf1.shape),
         _fullspec(Wf2.shape), _fullspec(bf2.shape)],
        out_specs=_rowspec(1),
        out_shape=jax.ShapeDtypeStruct((NP, 1), _f32),
    )(Q, Q, g4s, dis, W4, b4, Wf1, bf1, Wf2, bf2)


def kernel(x, edge_index, edge_attr, W1, b1, W2, b2, W3, b3, W4, b4,
           Wf1, bf1, Wf2, bf2):
    pad = EP - E
    src = jnp.concatenate([edge_index[0], jnp.zeros((pad,), _i32)])
    dst = jnp.concatenate([edge_index[1], jnp.zeros((pad,), _i32)])
    ew = jnp.concatenate([edge_attr, jnp.zeros((pad,), _f32)])

    src_es = src.reshape(NC * NS, NBLK_ES, KB)
    dst_es = dst.reshape(NC * NS, NBLK_ES, KB)
    ew_es = ew.reshape(NC * NS, NBLK_ES, KB)
    src_f128 = src.reshape(EP // KB, KB)
    dst_f128 = dst.reshape(EP // KB, KB)
    ew_f128 = ew.reshape(EP // KB, KB)
    src_f64 = src.reshape(2 * EP // KB, KB // 2)
    dst_f64 = dst.reshape(2 * EP // KB, KB // 2)
    ew_f64 = ew.reshape(2 * EP // KB, KB // 2)

    xp = jnp.pad(x, ((0, NP - N), (0, 0)))

    degp = _sc_deg(dst_es, ew_es).reshape(NC * NS, NP)
    dis, g1 = _tc_l1(xp, W1, degp)

    P = _sc_agg(g1, src_f128, dst_f128, ew_f128, 64, False, KB)
    g2 = _tc_post1(P, g1, dis, b1)

    P = _sc_agg(g2, src_f128, dst_f128, ew_f128, 64, False, KB)
    g3 = _tc_mid(P, g2, dis, W2, b2, 64, 128)

    P = _sc_agg(g3, src_f64, dst_f64, ew_f64, 128, False, KB // 2)
    g4s = _tc_l3(P, g3, dis, W3, b3)

    Q = _sc_agg(g4s.reshape(2 * NP, 128), src_f64, dst_f64, ew_f64, 128,
                True, KB // 2)
    out = _tc_head(Q, g4s, dis, W4, b4, Wf1, bf1, Wf2, bf2)
    return out[:N]


# speedup vs baseline: 1.0612x; 1.0550x over previous
import functools

import jax
import jax.numpy as jnp
from jax import lax
from jax.experimental import pallas as pl
from jax.experimental.pallas import tpu as pltpu
from jax.experimental.pallas import tpu_sc as plsc

N = 10000
E = 320000
D = 128
NP = 10240

NC = 2
NS = 16
L = 16
KB = 128
CH = 80

EP = 327680
NBLK_ES = EP // (NC * NS) // KB
NBLK_FS = EP // NS // KB
RA = NP // NS

RB = 512
G = NP // RB

_mesh = plsc.VectorSubcoreMesh(core_axis_name="c", subcore_axis_name="s",
                               num_cores=NC, num_subcores=NS)

_f32 = jnp.float32
_i32 = jnp.int32

_sc_params = pltpu.CompilerParams(needs_layout_passes=False,
                                  use_tc_tiling_on_sc=False)


def _splat16(v):
    return jnp.full((L,), v, dtype=_i32)


def _sc_deg(dst_es, ew_es):
    @functools.partial(
        pl.kernel,
        out_type=jax.ShapeDtypeStruct((NC * NS * NP,), _f32),
        mesh=_mesh,
        scratch_types=[
            pltpu.VMEM((NBLK_ES, KB), _i32),
            pltpu.VMEM((NBLK_ES, KB), _f32),
            pltpu.VMEM((NP,), _f32),
        ],
        compiler_params=_sc_params,
    )
    def k(dst_hbm, ew_hbm, out_hbm, dsts, ews, deg):
        c = lax.axis_index("c")
        s = lax.axis_index("s")
        wid = c * NS + s
        pltpu.sync_copy(dst_hbm.at[wid], dsts)
        pltpu.sync_copy(ew_hbm.at[wid], ews)

        @pl.loop(0, NP // L)
        def _(i):
            deg[pl.ds(i * L, L)] = jnp.zeros((L,), _f32)

        @pl.loop(0, NBLK_ES)
        def _(j):
            for k16 in range(KB // L):
                sl = pl.ds(k16 * L, L)
                plsc.addupdate_scatter(deg, [dsts[j, sl]], ews[j, sl])

        pltpu.sync_copy(deg, out_hbm.at[pl.ds(wid * NP, NP)])

    return k(dst_es, ew_es)


def _sc_agg(g, src_slab, dst_slab, ew_slab, w, feat_split, kb):
    A = EP // kb // (NC * NS)

    NB = 4
    NG = CH // NB

    @functools.partial(
        pl.kernel,
        out_type=jax.ShapeDtypeStruct((NC * NP, w), _f32),
        mesh=_mesh,
        scratch_types=[
            pltpu.VMEM((CH, kb), _i32),
            pltpu.VMEM((CH, kb), _i32),
            pltpu.VMEM((CH, kb), _f32),
        ] + [pltpu.VMEM((kb, w), _f32)] * NB
          + [pltpu.SemaphoreType.DMA] * (2 * NB)
          + [pltpu.VMEM_SHARED((NP, w), _f32)],
        compiler_params=_sc_params,
    )
    def k(g_hbm, src_hbm, dst_hbm, ew_hbm, out_hbm, srcs, dsts, ews, *rest):
        rows = rest[:NB]
        gs = rest[NB:2 * NB]
        ss = rest[2 * NB:3 * NB]
        acc = rest[3 * NB]
        c = lax.axis_index("c")
        s = lax.axis_index("s")
        if feat_split:
            base = s * (2 * A)
            nph = (2 * A) // CH
        else:
            base = (c * NS + s) * A
            nph = A // CH

        @pl.loop(0, kb)
        def _(e):
            for f in range(w // L):
                rows[0][e, pl.ds(f * L, L)] = jnp.zeros((L,), _f32)

        for i in range(RA // kb):
            pltpu.sync_copy(rows[0], acc.at[pl.ds(s * RA + i * kb, kb)])
        plsc.subcore_barrier()

        def gather(j, b):
            pltpu.make_async_copy(g_hbm.at[srcs.at[j]], rows[b], gs[b]).start()

        def gather_wait(j, b):
            pltpu.make_async_copy(g_hbm.at[srcs.at[j]], rows[b], gs[b]).wait()

        def scatter(j, b):
            pltpu.make_async_copy(rows[b], acc.at[dsts.at[j]],
                                  ss[b]).start(add=True)

        def scatter_wait(j, b):
            pltpu.make_async_copy(rows[b], acc.at[dsts.at[j]], ss[b]).wait()

        def scale(j, b):
            @plsc.parallel_loop(0, kb, unroll=8)
            def _(e):
                sp = plsc.load_gather(ews, [_splat16(j), _splat16(e)])
                for f in range(w // L):
                    sl = pl.ds(f * L, L)
                    rows[b][e, sl] = rows[b][e, sl] * sp

        @pl.loop(0, nph)
        def _(ph):
            pltpu.sync_copy(src_hbm.at[pl.ds(base + ph * CH, CH)], srcs)
            pltpu.sync_copy(dst_hbm.at[pl.ds(base + ph * CH, CH)], dsts)
            pltpu.sync_copy(ew_hbm.at[pl.ds(base + ph * CH, CH)], ews)

            if feat_split:
                off = jnp.full((L,), c * NP, dtype=_i32)

                @pl.loop(0, CH)
                def _(j):
                    for k16 in range(kb // L):
                        sl = pl.ds(k16 * L, L)
                        srcs[j, sl] = srcs[j, sl] + off

            for b in range(NB):
                gather(b, b)

            @pl.loop(0, NG)
            def _(g):
                for b in range(NB):
                    j = g * NB + b
                    gather_wait(j, b)
                    scale(j, b)
                    scatter(j, b)

                @pl.when(g < NG - 1)
                def _():
                    for b in range(NB):
                        j = g * NB + b
                        jn = j + NB
                        scatter_wait(j, b)
                        gather(jn, b)

            for b in range(NB):
                scatter_wait((NG - 1) * NB + b, b)

        plsc.subcore_barrier()
        pltpu.sync_copy(acc.at[pl.ds(s * RA, RA)],
                        out_hbm.at[pl.ds(c * NP + s * RA, RA)])

    return k(g, src_slab, dst_slab, ew_slab)


def _rowspec(w):
    return pl.BlockSpec((RB, w), lambda i: (i, 0))


def _fullspec(shape):
    nd = len(shape)
    return pl.BlockSpec(shape, lambda i, _nd=nd: (0,) * _nd)


def _dot_t(a, w):
    return lax.dot_general(a, w, (((1,), (1,)), ((), ())),
                           preferred_element_type=_f32)


def _tc_l1(x, W1, degp):
    def body(x_ref, w_ref, dp_ref, dis_ref, g1_ref):
        m = _dot_t(x_ref[...], w_ref[...])
        ones = jnp.ones((NC * NS, 1), _f32)
        d = 1.0 + lax.dot_general(dp_ref[...], ones, (((0,), (0,)), ((), ())),
                                  preferred_element_type=_f32)
        dis = lax.rsqrt(d)
        dis_ref[...] = dis
        g1_ref[...] = m * dis

    return pl.pallas_call(
        body,
        grid=(G,),
        in_specs=[_rowspec(D), _fullspec(W1.shape),
                  pl.BlockSpec((NC * NS, RB), lambda i: (0, i))],
        out_specs=[_rowspec(1), _rowspec(64)],
        out_shape=[jax.ShapeDtypeStruct((NP, 1), _f32),
                   jax.ShapeDtypeStruct((NP, 64), _f32)],
    )(x, W1, degp)


def _p_specs(w):
    return [pl.BlockSpec((RB, w), lambda i: (i, 0)),
            pl.BlockSpec((RB, w), lambda i: (G + i, 0))]


def _tc_post1(P, g1, dis, b1):
    def body(p0_ref, p1_ref, g1_ref, dis_ref, b_ref, g2_ref):
        dis = dis_ref[...]
        a = dis * (p0_ref[...] + p1_ref[...] + g1_ref[...])
        g2_ref[...] = dis * jnp.maximum(a + b_ref[...][None, :], 0.0)

    return pl.pallas_call(
        body,
        grid=(G,),
        in_specs=_p_specs(64) + [_rowspec(64), _rowspec(1),
                                 _fullspec(b1.shape)],
        out_specs=_rowspec(64),
        out_shape=jax.ShapeDtypeStruct((NP, 64), _f32),
    )(P, P, g1, dis, b1)


def _tc_mid(P, g, dis, W, b, win, wout):
    def body(p0_ref, p1_ref, g_ref, dis_ref, w_ref, b_ref, o_ref):
        dis = dis_ref[...]
        a = dis * (p0_ref[...] + p1_ref[...] + g_ref[...])
        h = jnp.maximum(_dot_t(a, w_ref[...]) + b_ref[...][None, :], 0.0)
        o_ref[...] = dis * h

    return pl.pallas_call(
        body,
        grid=(G,),
        in_specs=_p_specs(win) + [_rowspec(win), _rowspec(1),
                                  _fullspec(W.shape), _fullspec(b.shape)],
        out_specs=_rowspec(wout),
        out_shape=jax.ShapeDtypeStruct((NP, wout), _f32),
    )(P, P, g, dis, W, b)


def _tc_l3(P, g3, dis, W3, b3):
    def body(p0_ref, p1_ref, g_ref, dis_ref, w_ref, b_ref, o_ref):
        dis = dis_ref[...]
        a = dis * (p0_ref[...] + p1_ref[...] + g_ref[...])
        h = jnp.maximum(_dot_t(a, w_ref[...]) + b_ref[...][None, :], 0.0)
        g4 = dis * h
        o_ref[0] = g4[:, :128]
        o_ref[1] = g4[:, 128:]

    return pl.pallas_call(
        body,
        grid=(G,),
        in_specs=_p_specs(128) + [_rowspec(128), _rowspec(1),
                                  _fullspec(W3.shape), _fullspec(b3.shape)],
        out_specs=pl.BlockSpec((2, RB, 128), lambda i: (0, i, 0)),
        out_shape=jax.ShapeDtypeStruct((2, NP, 128), _f32),
    )(P, P, g3, dis, W3, b3)


def _tc_head(Q, g4s, dis, W4, b4, Wf1, bf1, Wf2, bf2):
    def body(q0_ref, q1_ref, g4_ref, dis_ref, w4_ref, b4_ref,
             wf1_ref, bf1_ref, wf2_ref, bf2_ref, o_ref):
        dis = dis_ref[...]
        a_lo = dis * (q0_ref[...] + g4_ref[0])
        a_hi = dis * (q1_ref[...] + g4_ref[1])
        a4 = jnp.concatenate([a_lo, a_hi], axis=1)
        h4 = jnp.maximum(_dot_t(a4, w4_ref[...]) + b4_ref[...][None, :], 0.0)
        z = jnp.maximum(_dot_t(h4, wf1_ref[...]) + bf1_ref[...][None, :], 0.0)
        o_ref[...] = (jnp.sum(z * wf2_ref[...], axis=1, keepdims=True)
                      + bf2_ref[0])

    return pl.pallas_call(
        body,
        grid=(G,),
        in_specs=_p_specs(128) +
        [pl.BlockSpec((2, RB, 128), lambda i: (0, i, 0)),
         _rowspec(1),
         _fullspec(W4.shape), _fullspec(b4.shape),
         _fullspec(Wf1.shape), _fullspec(bf1.shape),
         _fullspec(Wf2.shape), _fullspec(bf2.shape)],
        out_specs=_rowspec(1),
        out_shape=jax.ShapeDtypeStruct((NP, 1), _f32),
    )(Q, Q, g4s, dis, W4, b4, Wf1, bf1, Wf2, bf2)


def kernel(x, edge_index, edge_attr, W1, b1, W2, b2, W3, b3, W4, b4,
           Wf1, bf1, Wf2, bf2):
    pad = EP - E
    src = jnp.concatenate([edge_index[0], jnp.zeros((pad,), _i32)])
    dst = jnp.concatenate([edge_index[1], jnp.zeros((pad,), _i32)])
    ew = jnp.concatenate([edge_attr, jnp.zeros((pad,), _f32)])

    src_es = src.reshape(NC * NS, NBLK_ES, KB)
    dst_es = dst.reshape(NC * NS, NBLK_ES, KB)
    ew_es = ew.reshape(NC * NS, NBLK_ES, KB)
    src_f128 = src.reshape(EP // KB, KB)
    dst_f128 = dst.reshape(EP // KB, KB)
    ew_f128 = ew.reshape(EP // KB, KB)
    src_f64 = src.reshape(2 * EP // KB, KB // 2)
    dst_f64 = dst.reshape(2 * EP // KB, KB // 2)
    ew_f64 = ew.reshape(2 * EP // KB, KB // 2)

    xp = jnp.pad(x, ((0, NP - N), (0, 0)))

    degp = _sc_deg(dst_es, ew_es).reshape(NC * NS, NP)
    dis, g1 = _tc_l1(xp, W1, degp)

    P = _sc_agg(g1, src_f128, dst_f128, ew_f128, 64, False, KB)
    g2 = _tc_post1(P, g1, dis, b1)

    P = _sc_agg(g2, src_f128, dst_f128, ew_f128, 64, False, KB)
    g3 = _tc_mid(P, g2, dis, W2, b2, 64, 128)

    P = _sc_agg(g3, src_f64, dst_f64, ew_f64, 128, False, KB // 2)
    g4s = _tc_l3(P, g3, dis, W3, b3)

    Q = _sc_agg(g4s.reshape(2 * NP, 128), src_f64, dst_f64, ew_f64, 128,
                True, KB // 2)
    out = _tc_head(Q, g4s, dis, W4, b4, Wf1, bf1, Wf2, bf2)
    return out[:N]
